# baseline (speedup 1.0000x reference)
import functools

import jax
import jax.numpy as jnp
from jax import lax
from jax.experimental import pallas as pl
from jax.experimental.pallas import tpu as pltpu
from jax.experimental.pallas import tpu_sc as plsc

S = 819200
D = 64
DP = 128
NC = 2
NS = 16
NW = NC * NS
B_PER_W = S // NW
CHUNK = 256
N_CHUNKS = B_PER_W // CHUNK
NBUF = 2
NCB = 1
assert N_CHUNKS % NBUF == 0

_mesh = plsc.VectorSubcoreMesh(core_axis_name="c", subcore_axis_name="s")

_scratch = ([pltpu.VMEM((CHUNK,), jnp.int32) for _ in range(NBUF)]
            + [pltpu.VMEM((CHUNK, DP), jnp.float32) for _ in range(NBUF)]
            + [pltpu.VMEM((CHUNK, D), jnp.float32) for _ in range(NCB)]
            + [pltpu.SemaphoreType.DMA((NBUF,)),
               pltpu.SemaphoreType.DMA((NBUF,)),
               pltpu.SemaphoreType.DMA((NCB,))])


@functools.partial(
    pl.kernel,
    out_type=jax.ShapeDtypeStruct((S, D), jnp.float32),
    mesh=_mesh,
    scratch_types=_scratch,
)
def _gather_kernel(table_hbm, idx_hbm, out_hbm, *refs):
    idx_bufs = refs[:NBUF]
    row_bufs = refs[NBUF:2 * NBUF]
    c_bufs = refs[2 * NBUF:2 * NBUF + NCB]
    idx_sem, g_sem, o_sem = refs[2 * NBUF + NCB:]

    wid = lax.axis_index("s") * NC + lax.axis_index("c")
    base = wid * B_PER_W

    def start_idx(i, b):
        pltpu.async_copy(idx_hbm.at[pl.ds(base + i * CHUNK, CHUNK)],
                         idx_bufs[b], idx_sem.at[b])

    def wait_idx(b):
        pltpu.make_async_copy(idx_hbm.at[pl.ds(base, CHUNK)],
                              idx_bufs[b], idx_sem.at[b]).wait()

    def compact(b, cb):
        @pl.loop(0, CHUNK, unroll=4)
        def _row(r):
            for k in range(D // 16):
                c_bufs[cb][r, pl.ds(k * 16, 16)] = (
                    row_bufs[b][r, pl.ds(k * 16, 16)])

    def start_out(i, cb):
        off = base + i * CHUNK
        pltpu.async_copy(c_bufs[cb], out_hbm.at[pl.ds(off, CHUNK)],
                         o_sem.at[cb])

    def wait_out(cb):
        pltpu.make_async_copy(c_bufs[cb], out_hbm.at[pl.ds(0, CHUNK)],
                              o_sem.at[cb]).wait()

    for b in range(NBUF):
        start_idx(b, b)

    @pl.loop(0, N_CHUNKS, step=NBUF)
    def _group(g):
        for b in range(NBUF):
            wait_idx(b)
            pltpu.async_copy(table_hbm.at[idx_bufs[b]], row_bufs[b],
                             g_sem.at[b])
        for b in range(NBUF):
            i = g + b
            cb = b % NCB
            pltpu.make_async_copy(table_hbm.at[idx_bufs[b]], row_bufs[b],
                                  g_sem.at[b]).wait()

            @pl.when(i + NBUF < N_CHUNKS)
            def _():
                start_idx(i + NBUF, b)

            @pl.when(g + b >= NCB)
            def _():
                wait_out(cb)

            compact(b, cb)
            start_out(i, cb)

    for cb in range(NCB):
        wait_out(cb)


def kernel(flat_source, flat_idx):
    table128 = jnp.zeros(
        (flat_source.shape[0], DP), jnp.float32).at[:, :D].set(flat_source)
    return _gather_kernel(table128, flat_idx.astype(jnp.int32))


# speedup vs baseline: 1.2249x; 1.2249x over previous
import functools

import jax
import jax.numpy as jnp
from jax import lax
from jax.experimental import pallas as pl
from jax.experimental.pallas import tpu as pltpu
from jax.experimental.pallas import tpu_sc as plsc

S = 819200
D = 64
DP = 128
NC = 2
NS = 16
NW = NC * NS
B_PER_W = S // NW
CHUNK = 256
N_CHUNKS = B_PER_W // CHUNK
NBUF = 2
NCB = 1
assert N_CHUNKS % NBUF == 0

_mesh = plsc.VectorSubcoreMesh(core_axis_name="c", subcore_axis_name="s")

_scratch = ([pltpu.VMEM((CHUNK,), jnp.int32) for _ in range(NBUF)]
            + [pltpu.VMEM((CHUNK, DP), jnp.float32) for _ in range(NBUF)]
            + [pltpu.VMEM((CHUNK, D), jnp.float32) for _ in range(NCB)]
            + [pltpu.SemaphoreType.DMA((NBUF,)),
               pltpu.SemaphoreType.DMA((NBUF,)),
               pltpu.SemaphoreType.DMA((NCB,))])


@functools.partial(
    pl.kernel,
    out_type=jax.ShapeDtypeStruct((S, D), jnp.float32),
    mesh=_mesh,
    scratch_types=_scratch,
)
def _gather_kernel(table_hbm, idx_hbm, out_hbm, *refs):
    idx_bufs = refs[:NBUF]
    row_bufs = refs[NBUF:2 * NBUF]
    c_bufs = refs[2 * NBUF:2 * NBUF + NCB]
    idx_sem, g_sem, o_sem = refs[2 * NBUF + NCB:]

    wid = lax.axis_index("s") * NC + lax.axis_index("c")
    base = wid * B_PER_W

    def start_idx(i, b):
        pltpu.async_copy(idx_hbm.at[pl.ds(base + i * CHUNK, CHUNK)],
                         idx_bufs[b], idx_sem.at[b])

    def wait_idx(b):
        pltpu.make_async_copy(idx_hbm.at[pl.ds(base, CHUNK)],
                              idx_bufs[b], idx_sem.at[b]).wait()

    def compact(b, cb):
        @pl.loop(0, CHUNK, unroll=4)
        def _row(r):
            for k in range(D // 16):
                c_bufs[cb][r, pl.ds(k * 16, 16)] = (
                    row_bufs[b][r, pl.ds(k * 16, 16)])

    def start_out(i, cb):
        off = base + i * CHUNK
        pltpu.async_copy(c_bufs[cb], out_hbm.at[pl.ds(off, CHUNK)],
                         o_sem.at[cb])

    def wait_out(cb):
        pltpu.make_async_copy(c_bufs[cb], out_hbm.at[pl.ds(0, CHUNK)],
                              o_sem.at[cb]).wait()

    for b in range(NBUF):
        start_idx(b, b)

    @pl.loop(0, N_CHUNKS, step=NBUF)
    def _group(g):
        for b in range(NBUF):
            wait_idx(b)
            pltpu.async_copy(table_hbm.at[idx_bufs[b]], row_bufs[b],
                             g_sem.at[b])
        for b in range(NBUF):
            i = g + b
            cb = b % NCB
            pltpu.make_async_copy(table_hbm.at[idx_bufs[b]], row_bufs[b],
                                  g_sem.at[b]).wait()

            @pl.when(i + NBUF < N_CHUNKS)
            def _():
                start_idx(i + NBUF, b)

            @pl.when(g + b >= NCB)
            def _():
                wait_out(cb)

            compact(b, cb)
            start_out(i, cb)

    for cb in range(NCB):
        wait_out(cb)


def kernel(flat_source, flat_idx):
    table128 = jnp.concatenate(
        [flat_source, jnp.zeros((flat_source.shape[0], DP - D), jnp.float32)],
        axis=1)
    return _gather_kernel(table128, flat_idx.astype(jnp.int32))


# speedup vs baseline: 1.4054x; 1.1474x over previous
import functools

import jax
import jax.numpy as jnp
from jax import lax
from jax.experimental import pallas as pl
from jax.experimental.pallas import tpu as pltpu
from jax.experimental.pallas import tpu_sc as plsc

S = 819200
D = 64
NC = 2
NS = 16
NW = NC * NS
B_PER_W = S // NW
CHUNK = 512
N_CHUNKS = B_PER_W // CHUNK
NBUF = 2
assert N_CHUNKS % NBUF == 0

_mesh = plsc.VectorSubcoreMesh(core_axis_name="c", subcore_axis_name="s")


@functools.partial(
    pl.kernel,
    out_type=jax.ShapeDtypeStruct((S, D), jnp.float32),
    mesh=_mesh,
    scratch_types=[
        pltpu.VMEM((NBUF, CHUNK), jnp.int32),
        pltpu.VMEM((NBUF, CHUNK, D), jnp.float32),
        pltpu.SemaphoreType.DMA((NBUF,)),
        pltpu.SemaphoreType.DMA((NBUF,)),
        pltpu.SemaphoreType.DMA((NBUF,)),
    ],
    compiler_params=pltpu.CompilerParams(use_tc_tiling_on_sc=False),
)
def _gather_kernel(table_hbm, idx_hbm, out_hbm, idx_v, rows_v,
                   idx_sem, g_sem, o_sem):
    wid = lax.axis_index("s") * NC + lax.axis_index("c")
    base = wid * B_PER_W

    def start_idx(i, b):
        pltpu.async_copy(idx_hbm.at[pl.ds(base + i * CHUNK, CHUNK)],
                         idx_v.at[b], idx_sem.at[b])

    def wait_idx(b):
        pltpu.make_async_copy(idx_hbm.at[pl.ds(base, CHUNK)],
                              idx_v.at[b], idx_sem.at[b]).wait()

    def wait_out(b):
        pltpu.make_async_copy(rows_v.at[b],
                              out_hbm.at[pl.ds(base, CHUNK)], o_sem.at[b]).wait()

    for b in range(NBUF):
        start_idx(b, b)

    @pl.loop(0, N_CHUNKS, step=NBUF)
    def _group(g):
        for b in range(NBUF):
            wait_idx(b)

            @pl.when(g > 0)
            def _():
                wait_out(b)

            pltpu.async_copy(table_hbm.at[idx_v.at[b]], rows_v.at[b],
                             g_sem.at[b])
        for b in range(NBUF):
            i = g + b
            pltpu.make_async_copy(table_hbm.at[idx_v.at[b]], rows_v.at[b],
                                  g_sem.at[b]).wait()

            @pl.when(i + NBUF < N_CHUNKS)
            def _():
                start_idx(i + NBUF, b)

            pltpu.async_copy(rows_v.at[b],
                             out_hbm.at[pl.ds(base + i * CHUNK, CHUNK)],
                             o_sem.at[b])

    for b in range(NBUF):
        wait_out(b)


def kernel(flat_source, flat_idx):
    return _gather_kernel(flat_source, flat_idx.astype(jnp.int32))


# speedup vs baseline: 1.4089x; 1.0025x over previous
import functools

import jax
import jax.numpy as jnp
from jax import lax
from jax.experimental import pallas as pl
from jax.experimental.pallas import tpu as pltpu
from jax.experimental.pallas import tpu_sc as plsc

S = 819200
D = 64
NC = 2
NS = 16
NW = NC * NS
B_PER_W = S // NW
CHUNK = 256
N_CHUNKS = B_PER_W // CHUNK
NBUF = 4
assert N_CHUNKS % NBUF == 0

_mesh = plsc.VectorSubcoreMesh(core_axis_name="c", subcore_axis_name="s")


@functools.partial(
    pl.kernel,
    out_type=jax.ShapeDtypeStruct((S, D), jnp.float32),
    mesh=_mesh,
    scratch_types=[
        pltpu.VMEM((NBUF, CHUNK), jnp.int32),
        pltpu.VMEM((NBUF, CHUNK, D), jnp.float32),
        pltpu.SemaphoreType.DMA((NBUF,)),
        pltpu.SemaphoreType.DMA((NBUF,)),
        pltpu.SemaphoreType.DMA((NBUF,)),
    ],
    compiler_params=pltpu.CompilerParams(use_tc_tiling_on_sc=False),
)
def _gather_kernel(table_hbm, idx_hbm, out_hbm, idx_v, rows_v,
                   idx_sem, g_sem, o_sem):
    wid = lax.axis_index("s") * NC + lax.axis_index("c")
    base = wid * B_PER_W

    def start_idx(i, b):
        pltpu.async_copy(idx_hbm.at[pl.ds(base + i * CHUNK, CHUNK)],
                         idx_v.at[b], idx_sem.at[b])

    def wait_idx(b):
        pltpu.make_async_copy(idx_hbm.at[pl.ds(base, CHUNK)],
                              idx_v.at[b], idx_sem.at[b]).wait()

    def wait_out(b):
        pltpu.make_async_copy(rows_v.at[b],
                              out_hbm.at[pl.ds(base, CHUNK)], o_sem.at[b]).wait()

    for b in range(NBUF):
        start_idx(b, b)

    @pl.loop(0, N_CHUNKS, step=NBUF)
    def _group(g):
        for b in range(NBUF):
            wait_idx(b)

            @pl.when(g > 0)
            def _():
                wait_out(b)

            pltpu.async_copy(table_hbm.at[idx_v.at[b]], rows_v.at[b],
                             g_sem.at[b])
        for b in range(NBUF):
            i = g + b
            pltpu.make_async_copy(table_hbm.at[idx_v.at[b]], rows_v.at[b],
                                  g_sem.at[b]).wait()

            @pl.when(i + NBUF < N_CHUNKS)
            def _():
                start_idx(i + NBUF, b)

            pltpu.async_copy(rows_v.at[b],
                             out_hbm.at[pl.ds(base + i * CHUNK, CHUNK)],
                             o_sem.at[b])

    for b in range(NBUF):
        wait_out(b)


def kernel(flat_source, flat_idx):
    return _gather_kernel(flat_source, flat_idx.astype(jnp.int32))
